# baseline (speedup 1.0000x reference)
import jax
import jax.numpy as jnp
from jax import lax
from jax.experimental import pallas as pl
from jax.experimental.pallas import tpu as pltpu
from jax.experimental.pallas import tpu_sc as plsc

BATCH = 16384
RANK_K = 32
H1 = 256
H2 = 64
LW = 128
NUM_WORKERS = 32
UN = 1000000
VN = 100000
U_SPAN = UN // NUM_WORKERS
V_SPAN = VN // NUM_WORKERS
CHUNK = 1024
U_CHUNKS = 31
V_CHUNKS = 4
NVREG = BATCH // 16
OUT_ROWS = BATCH + 16


def _iota16():
    return lax.iota(jnp.int32, 16)


def _compact(stage_idx, hi, hp, my_lo, my_hi):
    lo_v = lax.broadcast_in_dim(my_lo, (16,), ())
    hi_v = lax.broadcast_in_dim(my_hi, (16,), ())
    ones = jnp.full((16,), 1, jnp.int32)
    zeros = jnp.full((16,), 0, jnp.int32)
    step16 = jnp.full((16,), 16, jnp.int32)
    c127 = jnp.full((16,), 127, jnp.int32)

    def body(k, hn):
        krow = lax.broadcast_in_dim(k >> 3, (16,), ())
        kcol = lax.broadcast_in_dim((k & 7) * 16, (16,), ()) + _iota16()
        vec = plsc.load_gather(stage_idx, [krow, kcol])
        mask = (vec >= lo_v) & (vec < hi_v)
        ones_m = jnp.where(mask, ones, zeros)
        pref = plsc.cumsum(ones_m)
        off = lax.broadcast_in_dim(hn, (16,), ()) + pref - ones
        pos = step16 * k + _iota16()
        plsc.store_scatter(hi, [lax.shift_right_logical(off, 7), off & c127],
                           vec, mask=mask)
        plsc.store_scatter(hp, [lax.shift_right_logical(off, 7), off & c127],
                           pos, mask=mask)
        return hn + jnp.sum(ones_m, axis=0)

    return lax.fori_loop(0, NVREG, body, jnp.int32(0))


def _scan_hits(ns, hn, hi, hp, buf, obuf, posarr, out_hbm, ssem, c_lo,
               width):
    zeros = jnp.full((16,), 0, jnp.int32)
    sent_base = jnp.full((16,), BATCH, jnp.int32)
    step16 = jnp.full((16,), 16, jnp.int32)
    lo_v = lax.broadcast_in_dim(c_lo, (16,), ())
    hiv = lax.broadcast_in_dim(c_lo + width, (16,), ())
    hn_v = lax.broadcast_in_dim(hn, (16,), ())
    nhv8 = lax.shift_right_logical(hn + 127, 7)

    def gbody(g8, ns):
        for u in range(8):
            g = g8 * 8 + u
            row = lax.broadcast_in_dim(g >> 3, (16,), ())
            col = lax.broadcast_in_dim((g & 7) * 16, (16,), ()) + _iota16()
            iv = plsc.load_gather(hi, [row, col])
            pv = plsc.load_gather(hp, [row, col])
            e = step16 * g + _iota16()
            m = (e < hn_v) & (iv >= lo_v) & (iv < hiv)

            def extract(ns, m=m, iv=iv, pv=pv):
                def wait_prev(n):
                    pltpu.make_async_copy(obuf, out_hbm.at[posarr.at[0]],
                                          ssem).wait()
                    return n

                ns = lax.cond(((ns & 7) == 0) & (ns >= 8), wait_prev,
                              lambda n: n, ns)
                rr = jnp.where(m, iv - lo_v, zeros)
                pos = jnp.where(m, pv, sent_base + _iota16())
                srow = lax.broadcast_in_dim((ns & 7) * 16, (16,), ())
                orow = srow + _iota16()
                for cf in range(RANK_K):
                    cvec = jnp.full((16,), cf, jnp.int32)
                    vals = plsc.load_gather(buf, [cvec, rr])
                    plsc.store_scatter(obuf, [orow, cvec], vals)
                plsc.store_scatter(posarr, [zeros, srow + _iota16()], pos)

                def flush(n):
                    pltpu.async_copy(obuf, out_hbm.at[posarr.at[0]], ssem)
                    return n

                return lax.cond((ns & 7) == 7, flush, lambda n: n, ns + 1)

            ns = lax.cond(jnp.any(m), extract, lambda n: n, ns)
        return ns

    return lax.fori_loop(0, nhv8, gbody, ns)


def _drain(ns, hn, hi, hp, buf, obuf, posarr, out_hbm, ssem):
    zeros = jnp.full((16,), 0, jnp.int32)
    sent = jnp.full((16,), BATCH, jnp.int32) + _iota16()
    npad = 8 - (ns & 7)

    def pbody(p, ns):
        def wait_prev(n):
            pltpu.make_async_copy(obuf, out_hbm.at[posarr.at[0]],
                                  ssem).wait()
            return n

        ns = lax.cond(((ns & 7) == 0) & (ns >= 8), wait_prev,
                      lambda n: n, ns)
        srow = lax.broadcast_in_dim((ns & 7) * 16, (16,), ())
        plsc.store_scatter(posarr, [zeros, srow + _iota16()], sent)

        def flush(n):
            pltpu.async_copy(obuf, out_hbm.at[posarr.at[0]], ssem)
            return n

        return lax.cond((ns & 7) == 7, flush, lambda n: n, ns + 1)

    ns = lax.fori_loop(0, npad, pbody, ns)
    pltpu.make_async_copy(obuf, out_hbm.at[posarr.at[0]], ssem).wait()
    return jnp.int32(0)


def _extract_phase(tab, tail_in, out_hbm, stage_idx, li, lp, buf0,
                   tailbuf, obuf, posarr, ssem, my_lo, my_hi,
                   n_chunks, table_n, tail_w):
    hn = _compact(stage_idx, li, lp, my_lo, my_hi)
    tail_lo = (table_n // 128) * 128
    clamp = (table_n - CHUNK) & ~127
    zeros = jnp.full((16,), 0, jnp.int32)
    sent = jnp.full((16,), BATCH, jnp.int32) + _iota16()
    for s in range(8):
        plsc.store_scatter(posarr, [zeros, jnp.full((16,), s * 16, jnp.int32)
                                    + _iota16()], sent)

    def chunk_body(c, ns):
        a = (my_lo + c * CHUNK) & ~127
        c_lo = pl.multiple_of(jnp.minimum(a, clamp), 128)
        pltpu.sync_copy(tab.at[:, pl.ds(c_lo, CHUNK)], buf0)
        return _scan_hits(ns, hn, li, lp, buf0, obuf, posarr, out_hbm, ssem,
                          c_lo, CHUNK)

    ns = lax.fori_loop(0, n_chunks, chunk_body, jnp.int32(0))
    if tail_w:
        pltpu.sync_copy(tail_in, tailbuf)
        ns = _scan_hits(ns, hn, li, lp, tailbuf, obuf, posarr, out_hbm, ssem,
                        jnp.int32(tail_lo), tail_w)
    _drain(ns, hn, li, lp, buf0, obuf, posarr, out_hbm, ssem)


def _gather_body(ut, vt, ut_tail, vt_tail, i_hbm, j_hbm, u_out, v_out,
                 stage_idx, li, lp, buf0, tailbuf, obuf, posarr, ssem):
    wid = lax.axis_index("s") * 2 + lax.axis_index("c")
    zero16 = jnp.zeros((16,), jnp.float32)

    def zbody(r, c):
        rv = lax.broadcast_in_dim(r, (16,), ())
        for cc in range(RANK_K, LW, 16):
            cv = jnp.full((16,), cc, jnp.int32) + _iota16()
            plsc.store_scatter(obuf, [rv, cv], zero16)
        return c

    lax.fori_loop(0, 128, zbody, jnp.int32(0))
    pltpu.sync_copy(i_hbm, stage_idx)
    _extract_phase(ut, ut_tail, u_out, stage_idx, li, lp, buf0,
                   tailbuf, obuf, posarr, ssem,
                   wid * U_SPAN, (wid + 1) * U_SPAN, U_CHUNKS, UN,
                   UN - (UN // 128) * 128)
    pltpu.sync_copy(j_hbm, stage_idx)
    _extract_phase(vt, vt_tail, v_out, stage_idx, li, lp, buf0,
                   tailbuf, obuf, posarr, ssem,
                   wid * V_SPAN, (wid + 1) * V_SPAN, V_CHUNKS, VN,
                   VN - (VN // 128) * 128)


def _make_gather():
    mesh = plsc.VectorSubcoreMesh(core_axis_name="c", subcore_axis_name="s")
    return pl.kernel(
        _gather_body,
        out_type=(
            jax.ShapeDtypeStruct((OUT_ROWS, LW), jnp.float32),
            jax.ShapeDtypeStruct((OUT_ROWS, LW), jnp.float32),
        ),
        mesh=mesh,
        scratch_types=[
            pltpu.VMEM((128, 128), jnp.int32),
            pltpu.VMEM((128, 128), jnp.int32),
            pltpu.VMEM((128, 128), jnp.int32),
            pltpu.VMEM((RANK_K, CHUNK), jnp.float32),
            pltpu.VMEM((RANK_K, 128), jnp.float32),
            pltpu.VMEM((128, LW), jnp.float32),
            pltpu.VMEM((1, 128), jnp.int32),
            pltpu.SemaphoreType.DMA,
        ],
        compiler_params=pltpu.CompilerParams(needs_layout_passes=False),
    )


def _mlp_body(u_ref, v_ref, w1u_ref, w1v_ref, w2_ref, wl_ref, out_ref):
    h = lax.dot_general(u_ref[...], w1u_ref[...], (((1,), (1,)), ((), ())),
                        preferred_element_type=jnp.float32)
    h = h + lax.dot_general(v_ref[...], w1v_ref[...], (((1,), (1,)), ((), ())),
                            preferred_element_type=jnp.float32)
    h = 0.5 * h * (1.0 + lax.erf(h * 0.7071067811865476))
    y = lax.dot_general(h, w2_ref[...], (((1,), (1,)), ((), ())),
                        preferred_element_type=jnp.float32)
    out_ref[...] = jnp.sum(y * wl_ref[...], axis=1)


def _make_mlp(bb):
    return pl.pallas_call(
        _mlp_body,
        grid=(BATCH // bb,),
        in_specs=[
            pl.BlockSpec((bb, LW), lambda b: (b, 0)),
            pl.BlockSpec((bb, LW), lambda b: (b, 0)),
            pl.BlockSpec((H1, LW), lambda b: (0, 0)),
            pl.BlockSpec((H1, LW), lambda b: (0, 0)),
            pl.BlockSpec((H2, H1), lambda b: (0, 0)),
            pl.BlockSpec((1, H2), lambda b: (0, 0)),
        ],
        out_specs=pl.BlockSpec((bb,), lambda b: (b,)),
        out_shape=jax.ShapeDtypeStruct((BATCH,), jnp.float32),
    )


def kernel(i, j, U, V, W1, W2, Wl):
    i = i.astype(jnp.int32)
    j = j.astype(jnp.int32)
    ut = U.T
    vt = V.T
    ut_tail = jnp.pad(ut[:, (UN // 128) * 128:], ((0, 0), (0, 128 - UN % 128)))
    vt_tail = jnp.pad(vt[:, (VN // 128) * 128:], ((0, 0), (0, 128 - VN % 128)))
    u128, v128 = _make_gather()(ut, vt, ut_tail, vt_tail,
                                i.reshape(128, 128), j.reshape(128, 128))
    w1u = jnp.pad(W1[:, :RANK_K], ((0, 0), (0, LW - RANK_K)))
    w1v = jnp.pad(W1[:, RANK_K:], ((0, 0), (0, LW - RANK_K)))
    return _make_mlp(2048)(u128, v128, w1u, w1v, W2, Wl)


# speedup vs baseline: 2.0505x; 2.0505x over previous
import jax
import jax.numpy as jnp
from jax import lax
from jax.experimental import pallas as pl
from jax.experimental.pallas import tpu as pltpu
from jax.experimental.pallas import tpu_sc as plsc

BATCH = 16384
RANK_K = 32
H1 = 256
H2 = 64
LW = 128
NUM_WORKERS = 32
UN = 1000000
VN = 100000
U_SPAN = UN // NUM_WORKERS
V_SPAN = VN // NUM_WORKERS
CHUNK = 1024
U_CHUNKS = 31
V_CHUNKS = 4
NVREG = BATCH // 16
OUT_ROWS = BATCH + 16 * NUM_WORKERS


def _iota16():
    return lax.iota(jnp.int32, 16)


def _compact(stage_idx, hi, hp, my_lo, my_hi, sent0=BATCH):
    lo_v = lax.broadcast_in_dim(my_lo, (16,), ())
    hi_v = lax.broadcast_in_dim(my_hi, (16,), ())
    ones = jnp.full((16,), 1, jnp.int32)
    zeros = jnp.full((16,), 0, jnp.int32)
    step16 = jnp.full((16,), 16, jnp.int32)
    c127 = jnp.full((16,), 127, jnp.int32)

    def body(k, hn):
        krow = lax.broadcast_in_dim(k >> 3, (16,), ())
        kcol = lax.broadcast_in_dim((k & 7) * 16, (16,), ()) + _iota16()
        vec = plsc.load_gather(stage_idx, [krow, kcol])
        mask = (vec >= lo_v) & (vec < hi_v)
        ones_m = jnp.where(mask, ones, zeros)
        pref = plsc.cumsum(ones_m)
        off = lax.broadcast_in_dim(hn, (16,), ()) + pref - ones
        pos = step16 * k + _iota16()
        plsc.store_scatter(hi, [lax.shift_right_logical(off, 7), off & c127],
                           vec, mask=mask)
        plsc.store_scatter(hp, [lax.shift_right_logical(off, 7), off & c127],
                           pos, mask=mask)
        return hn + jnp.sum(ones_m, axis=0)

    return lax.fori_loop(0, NVREG, body, jnp.int32(0))


def _scan_hits(ns, hn, hi, hp, buf, obuf, posarr, out_hbm, ssem, c_lo,
               width, sent0=BATCH):
    zeros = jnp.full((16,), 0, jnp.int32)
    sent_base = lax.broadcast_in_dim(sent0, (16,), ())
    step16 = jnp.full((16,), 16, jnp.int32)
    lo_v = lax.broadcast_in_dim(c_lo, (16,), ())
    hiv = lax.broadcast_in_dim(c_lo + width, (16,), ())
    hn_v = lax.broadcast_in_dim(hn, (16,), ())
    nhv8 = lax.shift_right_logical(hn + 127, 7)

    def gbody(g8, ns):
        for u in range(8):
            g = g8 * 8 + u
            row = lax.broadcast_in_dim(g >> 3, (16,), ())
            col = lax.broadcast_in_dim((g & 7) * 16, (16,), ()) + _iota16()
            iv = plsc.load_gather(hi, [row, col])
            pv = plsc.load_gather(hp, [row, col])
            e = step16 * g + _iota16()
            m = (e < hn_v) & (iv >= lo_v) & (iv < hiv)

            def extract(ns, m=m, iv=iv, pv=pv):
                def wait_prev(n):
                    pltpu.make_async_copy(obuf, out_hbm.at[posarr.at[0]],
                                          ssem).wait()
                    return n

                ns = lax.cond(((ns & 7) == 0) & (ns >= 8), wait_prev,
                              lambda n: n, ns)
                rr = jnp.where(m, iv - lo_v, zeros)
                pos = jnp.where(m, pv, sent_base + _iota16())
                srow = lax.broadcast_in_dim((ns & 7) * 16, (16,), ())
                orow = srow + _iota16()
                for cf in range(RANK_K):
                    cvec = jnp.full((16,), cf, jnp.int32)
                    vals = plsc.load_gather(buf, [cvec, rr])
                    plsc.store_scatter(obuf, [orow, cvec], vals)
                plsc.store_scatter(posarr, [zeros, srow + _iota16()], pos)

                def flush(n):
                    pltpu.async_copy(obuf, out_hbm.at[posarr.at[0]], ssem)
                    return n

                return lax.cond((ns & 7) == 7, flush, lambda n: n, ns + 1)

            ns = lax.cond(jnp.any(m), extract, lambda n: n, ns)
        return ns

    return lax.fori_loop(0, nhv8, gbody, ns)


def _drain(ns, hn, hi, hp, buf, obuf, posarr, out_hbm, ssem, sent0=BATCH):
    zeros = jnp.full((16,), 0, jnp.int32)
    sent = lax.broadcast_in_dim(sent0, (16,), ()) + _iota16()
    npad = 8 - (ns & 7)

    def pbody(p, ns):
        def wait_prev(n):
            pltpu.make_async_copy(obuf, out_hbm.at[posarr.at[0]],
                                  ssem).wait()
            return n

        ns = lax.cond(((ns & 7) == 0) & (ns >= 8), wait_prev,
                      lambda n: n, ns)
        srow = lax.broadcast_in_dim((ns & 7) * 16, (16,), ())
        plsc.store_scatter(posarr, [zeros, srow + _iota16()], sent)

        def flush(n):
            pltpu.async_copy(obuf, out_hbm.at[posarr.at[0]], ssem)
            return n

        return lax.cond((ns & 7) == 7, flush, lambda n: n, ns + 1)

    ns = lax.fori_loop(0, npad, pbody, ns)
    pltpu.make_async_copy(obuf, out_hbm.at[posarr.at[0]], ssem).wait()
    return jnp.int32(0)


def _extract_phase(tab, tail_in, out_hbm, stage_idx, li, lp, buf0,
                   tailbuf, obuf, posarr, ssem, my_lo, my_hi,
                   n_chunks, table_n, tail_w, sent0=BATCH):
    hn = _compact(stage_idx, li, lp, my_lo, my_hi)
    tail_lo = (table_n // 128) * 128
    clamp = (table_n - CHUNK) & ~127
    zeros = jnp.full((16,), 0, jnp.int32)
    sent = lax.broadcast_in_dim(sent0, (16,), ()) + _iota16()
    for s in range(8):
        plsc.store_scatter(posarr, [zeros, jnp.full((16,), s * 16, jnp.int32)
                                    + _iota16()], sent)

    def chunk_body(c, ns):
        a = (my_lo + c * CHUNK) & ~127
        c_lo = pl.multiple_of(jnp.minimum(a, clamp), 128)
        pltpu.sync_copy(tab.at[:, pl.ds(c_lo, CHUNK)], buf0)
        return _scan_hits(ns, hn, li, lp, buf0, obuf, posarr, out_hbm, ssem,
                          c_lo, CHUNK, sent0)

    ns = lax.fori_loop(0, n_chunks, chunk_body, jnp.int32(0))
    if tail_w:
        pltpu.sync_copy(tail_in, tailbuf)
        ns = _scan_hits(ns, hn, li, lp, tailbuf, obuf, posarr, out_hbm, ssem,
                        jnp.int32(tail_lo), tail_w, sent0)
    _drain(ns, hn, li, lp, buf0, obuf, posarr, out_hbm, ssem, sent0)


def _gather_body(ut, vt, ut_tail, vt_tail, i_hbm, j_hbm, u_out, v_out,
                 stage_idx, li, lp, buf0, tailbuf, obuf, posarr, ssem):
    wid = lax.axis_index("s") * 2 + lax.axis_index("c")
    zero16 = jnp.zeros((16,), jnp.float32)

    def zbody(r, c):
        rv = lax.broadcast_in_dim(r, (16,), ())
        for cc in range(RANK_K, LW, 16):
            cv = jnp.full((16,), cc, jnp.int32) + _iota16()
            plsc.store_scatter(obuf, [rv, cv], zero16)
        return c

    lax.fori_loop(0, 128, zbody, jnp.int32(0))
    pltpu.sync_copy(i_hbm, stage_idx)
    sent0 = BATCH + wid * 16
    _extract_phase(ut, ut_tail, u_out, stage_idx, li, lp, buf0,
                   tailbuf, obuf, posarr, ssem,
                   wid * U_SPAN, (wid + 1) * U_SPAN, U_CHUNKS, UN,
                   UN - (UN // 128) * 128, sent0)
    pltpu.sync_copy(j_hbm, stage_idx)
    _extract_phase(vt, vt_tail, v_out, stage_idx, li, lp, buf0,
                   tailbuf, obuf, posarr, ssem,
                   wid * V_SPAN, (wid + 1) * V_SPAN, V_CHUNKS, VN,
                   VN - (VN // 128) * 128, sent0)


def _make_gather():
    mesh = plsc.VectorSubcoreMesh(core_axis_name="c", subcore_axis_name="s")
    return pl.kernel(
        _gather_body,
        out_type=(
            jax.ShapeDtypeStruct((OUT_ROWS, LW), jnp.float32),
            jax.ShapeDtypeStruct((OUT_ROWS, LW), jnp.float32),
        ),
        mesh=mesh,
        scratch_types=[
            pltpu.VMEM((128, 128), jnp.int32),
            pltpu.VMEM((128, 128), jnp.int32),
            pltpu.VMEM((128, 128), jnp.int32),
            pltpu.VMEM((RANK_K, CHUNK), jnp.float32),
            pltpu.VMEM((RANK_K, 128), jnp.float32),
            pltpu.VMEM((128, LW), jnp.float32),
            pltpu.VMEM((1, 128), jnp.int32),
            pltpu.SemaphoreType.DMA,
        ],
        compiler_params=pltpu.CompilerParams(needs_layout_passes=False),
    )


def _mlp_body(u_ref, v_ref, w1u_ref, w1v_ref, w2_ref, wl_ref, out_ref):
    h = lax.dot_general(u_ref[...], w1u_ref[...], (((1,), (1,)), ((), ())),
                        preferred_element_type=jnp.float32)
    h = h + lax.dot_general(v_ref[...], w1v_ref[...], (((1,), (1,)), ((), ())),
                            preferred_element_type=jnp.float32)
    h = 0.5 * h * (1.0 + lax.erf(h * 0.7071067811865476))
    y = lax.dot_general(h, w2_ref[...], (((1,), (1,)), ((), ())),
                        preferred_element_type=jnp.float32)
    out_ref[...] = jnp.sum(y * wl_ref[...], axis=1)


def _make_mlp(bb):
    return pl.pallas_call(
        _mlp_body,
        grid=(BATCH // bb,),
        in_specs=[
            pl.BlockSpec((bb, LW), lambda b: (b, 0)),
            pl.BlockSpec((bb, LW), lambda b: (b, 0)),
            pl.BlockSpec((H1, LW), lambda b: (0, 0)),
            pl.BlockSpec((H1, LW), lambda b: (0, 0)),
            pl.BlockSpec((H2, H1), lambda b: (0, 0)),
            pl.BlockSpec((1, H2), lambda b: (0, 0)),
        ],
        out_specs=pl.BlockSpec((bb,), lambda b: (b,)),
        out_shape=jax.ShapeDtypeStruct((BATCH,), jnp.float32),
    )


def kernel(i, j, U, V, W1, W2, Wl):
    i = i.astype(jnp.int32)
    j = j.astype(jnp.int32)
    ut = U.T
    vt = V.T
    ut_tail = jnp.pad(ut[:, (UN // 128) * 128:], ((0, 0), (0, 128 - UN % 128)))
    vt_tail = jnp.pad(vt[:, (VN // 128) * 128:], ((0, 0), (0, 128 - VN % 128)))
    u128, v128 = _make_gather()(ut, vt, ut_tail, vt_tail,
                                i.reshape(128, 128), j.reshape(128, 128))
    w1u = jnp.pad(W1[:, :RANK_K], ((0, 0), (0, LW - RANK_K)))
    w1v = jnp.pad(W1[:, RANK_K:], ((0, 0), (0, LW - RANK_K)))
    return _make_mlp(2048)(u128, v128, w1u, w1v, W2, Wl)
